# baseline (speedup 1.0000x reference)
import functools

import jax
import jax.numpy as jnp
from jax import lax
from jax.experimental import pallas as pl
from jax.experimental.pallas import tpu as pltpu
from jax.experimental.pallas import tpu_sc as plsc


def _sc_geometry():
    try:
        info = plsc.get_sparse_core_info()
        return info.num_cores, info.num_subcores
    except Exception:
        return 2, 16


def _embed_lookup(x2d, table, pe):
    B, S = x2d.shape
    V, D = table.shape
    NC, NS = _sc_geometry()
    NW = NC * NS
    C = S // NW

    mesh = plsc.VectorSubcoreMesh(core_axis_name="c", subcore_axis_name="s")

    @functools.partial(
        pl.kernel,
        mesh=mesh,
        out_type=jax.ShapeDtypeStruct((B, S, D), jnp.float32),
        scratch_types=[
            pltpu.VMEM((C,), jnp.int32),
            pltpu.VMEM((C, D), jnp.float32),
            pltpu.VMEM((C, D), jnp.float32),
            pltpu.SemaphoreType.DMA,
        ],
    )
    def emb(x_hbm, table_hbm, pe_hbm, out_hbm, idx_v, rows_v, pe_v, sem):
        wid = lax.axis_index("s") * NC + lax.axis_index("c")
        base = wid * C
        pltpu.sync_copy(pe_hbm.at[pl.ds(base, C)], pe_v)
        nj = D // 16

        def add_pe_row(r, carry):
            for j in range(nj):
                plsc.addupdate(rows_v.at[r, pl.ds(j * 16, 16)],
                               pe_v[r, pl.ds(j * 16, 16)])
            return carry

        for b in range(B):
            pltpu.sync_copy(x_hbm.at[b, pl.ds(base, C)], idx_v)
            pltpu.async_copy(table_hbm.at[idx_v], rows_v, sem).wait()
            lax.fori_loop(0, C, add_pe_row, 0)
            pltpu.sync_copy(rows_v, out_hbm.at[b, pl.ds(base, C)])

    return emb(x2d, table, pe)


def kernel(x, table, pe):
    return _embed_lookup(x.astype(jnp.int32), table, pe.astype(jnp.float32))
